# baseline (speedup 1.0000x reference)
import functools

import jax
import jax.numpy as jnp
from jax import lax
from jax.experimental import pallas as pl
from jax.experimental.pallas import tpu as pltpu
from jax.experimental.pallas import tpu_sc as plsc

_VOCAB = 256
_EMB = 32
_B = 4096
_L = 200
_N = _B * _L
_NC = 2
_NS = 16
_NW = _NC * _NS
_N_PER_W = _N // _NW
_CHUNK = 1600
_N_CHUNKS = _N_PER_W // _CHUNK
_GROUPS = _CHUNK // 16

_mesh = plsc.VectorSubcoreMesh(core_axis_name="c", subcore_axis_name="s")


@functools.partial(
    pl.kernel,
    out_type=jax.ShapeDtypeStruct((_N * _EMB,), jnp.float32),
    mesh=_mesh,
    scratch_types=[
        pltpu.VMEM((_VOCAB * _EMB,), jnp.float32),
        pltpu.VMEM((2, _CHUNK), jnp.int32),
        pltpu.VMEM((2, _CHUNK * _EMB), jnp.float32),
        pltpu.SemaphoreType.DMA((2,)),
        pltpu.SemaphoreType.DMA((2,)),
    ],
    compiler_params=pltpu.CompilerParams(use_tc_tiling_on_sc=False,
                                         needs_layout_passes=False),
)
def _gather_kernel(ids_hbm, table_hbm, out_hbm, table_v, idx_v, rows_v,
                   sem_idx, sem_w):
    wid = lax.axis_index("s") * _NC + lax.axis_index("c")
    base = wid * _N_PER_W

    pltpu.sync_copy(table_hbm, table_v)

    lane = lax.iota(jnp.int32, 16)
    st_lane = lane * _EMB

    for s in range(2):
        pltpu.async_copy(ids_hbm.at[pl.ds(base + s * _CHUNK, _CHUNK)],
                         idx_v.at[s], sem_idx.at[s])

    @pl.loop(0, _N_CHUNKS, step=2)
    def _steady(i):
        for s in range(2):
            c = i + s
            off = base + c * _CHUNK
            pltpu.make_async_copy(ids_hbm.at[pl.ds(off, _CHUNK)],
                                  idx_v.at[s], sem_idx.at[s]).wait()

            @pl.when(c >= 2)
            def _():
                pltpu.make_async_copy(
                    rows_v.at[s],
                    out_hbm.at[pl.ds((off - 2 * _CHUNK) * _EMB,
                                     _CHUNK * _EMB)],
                    sem_w.at[s]).wait()

            idx_ref = idx_v.at[s]
            rows_ref = rows_v.at[s]

            @pl.loop(0, _GROUPS)
            def _group(g):
                idvec = plsc.load_gather(idx_ref, [g * 16 + lane])
                gaddr = idvec * _EMB
                staddr = g * (16 * _EMB) + st_lane
                for e in range(_EMB):
                    val = plsc.load_gather(table_v, [gaddr + e])
                    plsc.store_scatter(rows_ref, [staddr + e], val)

            pltpu.async_copy(rows_v.at[s],
                             out_hbm.at[pl.ds(off * _EMB, _CHUNK * _EMB)],
                             sem_w.at[s])

            @pl.when(c + 2 < _N_CHUNKS)
            def _():
                pltpu.async_copy(
                    ids_hbm.at[pl.ds(off + 2 * _CHUNK, _CHUNK)],
                    idx_v.at[s], sem_idx.at[s])

    for s in range(2):
        off = base + (_N_CHUNKS - 2 + s) * _CHUNK
        pltpu.make_async_copy(rows_v.at[s],
                              out_hbm.at[pl.ds(off * _EMB, _CHUNK * _EMB)],
                              sem_w.at[s]).wait()


def kernel(char_ids, char_embs):
    ids_flat = char_ids.reshape(_N)
    out = _gather_kernel(ids_flat, char_embs.reshape(_VOCAB * _EMB))
    return out.reshape(_B, _L, _EMB)


# speedup vs baseline: 1.2016x; 1.2016x over previous
import functools

import jax
import jax.numpy as jnp
from jax import lax
from jax.experimental import pallas as pl
from jax.experimental.pallas import tpu as pltpu
from jax.experimental.pallas import tpu_sc as plsc

_VOCAB = 256
_EMB = 32
_B = 4096
_L = 200
_N = _B * _L
_NC = 2
_NS = 16
_NW = _NC * _NS
_N_PER_W = _N // _NW
_CHUNK = 1600
_N_CHUNKS = _N_PER_W // _CHUNK
_GROUPS = _CHUNK // 16

_mesh = plsc.VectorSubcoreMesh(core_axis_name="c", subcore_axis_name="s")


@functools.partial(
    pl.kernel,
    out_type=jax.ShapeDtypeStruct((_N * _EMB,), jnp.float32),
    mesh=_mesh,
    scratch_types=[
        pltpu.VMEM((_VOCAB * _EMB,), jnp.float32),
        pltpu.VMEM((2, _CHUNK), jnp.int32),
        pltpu.VMEM((2, _CHUNK * _EMB), jnp.float32),
        pltpu.SemaphoreType.DMA((2,)),
        pltpu.SemaphoreType.DMA((2,)),
    ],
    compiler_params=pltpu.CompilerParams(use_tc_tiling_on_sc=False,
                                         needs_layout_passes=False),
)
def _gather_kernel(ids_hbm, table_hbm, out_hbm, table_v, idx_v, rows_v,
                   sem_idx, sem_w):
    wid = lax.axis_index("s") * _NC + lax.axis_index("c")
    base = wid * _N_PER_W

    pltpu.sync_copy(table_hbm, table_v)

    lane = lax.iota(jnp.int32, 16)
    st_lane = lane * _EMB

    for s in range(2):
        pltpu.async_copy(ids_hbm.at[pl.ds(base + s * _CHUNK, _CHUNK)],
                         idx_v.at[s], sem_idx.at[s])

    @pl.loop(0, _N_CHUNKS, step=2)
    def _steady(i):
        for s in range(2):
            c = i + s
            off = base + c * _CHUNK
            pltpu.make_async_copy(ids_hbm.at[pl.ds(off, _CHUNK)],
                                  idx_v.at[s], sem_idx.at[s]).wait()

            @pl.when(c >= 2)
            def _():
                pltpu.make_async_copy(
                    rows_v.at[s],
                    out_hbm.at[pl.ds((off - 2 * _CHUNK) * _EMB,
                                     _CHUNK * _EMB)],
                    sem_w.at[s]).wait()

            idx_ref = idx_v.at[s]
            rows_ref = rows_v.at[s]

            @plsc.parallel_loop(0, _GROUPS, unroll=2)
            def _group(g):
                idvec = plsc.load_gather(idx_ref, [g * 16 + lane])
                gaddr = idvec * _EMB
                staddr = g * (16 * _EMB) + st_lane
                for e in range(_EMB):
                    val = plsc.load_gather(table_v, [gaddr + e])
                    plsc.store_scatter(rows_ref, [staddr + e], val)

            pltpu.async_copy(rows_v.at[s],
                             out_hbm.at[pl.ds(off * _EMB, _CHUNK * _EMB)],
                             sem_w.at[s])

            @pl.when(c + 2 < _N_CHUNKS)
            def _():
                pltpu.async_copy(
                    ids_hbm.at[pl.ds(off + 2 * _CHUNK, _CHUNK)],
                    idx_v.at[s], sem_idx.at[s])

    for s in range(2):
        off = base + (_N_CHUNKS - 2 + s) * _CHUNK
        pltpu.make_async_copy(rows_v.at[s],
                              out_hbm.at[pl.ds(off * _EMB, _CHUNK * _EMB)],
                              sem_w.at[s]).wait()


def kernel(char_ids, char_embs):
    ids_flat = char_ids.reshape(_N)
    out = _gather_kernel(ids_flat, char_embs.reshape(_VOCAB * _EMB))
    return out.reshape(_B, _L, _EMB)


# speedup vs baseline: 2.3376x; 1.9453x over previous
import functools

import jax
import jax.numpy as jnp
from jax import lax
from jax.experimental import pallas as pl
from jax.experimental.pallas import tpu as pltpu
from jax.experimental.pallas import tpu_sc as plsc

_VOCAB = 256
_EMB = 32
_B = 4096
_L = 200
_N = _B * _L
_NC = 2
_NS = 16
_NW = _NC * _NS
_N_PER_W = _N // _NW
_CHUNK = 1600
_N_CHUNKS = _N_PER_W // _CHUNK
_Q = 1088
_RGROUPS = (_CHUNK - _Q) // 16

_mesh = plsc.VectorSubcoreMesh(core_axis_name="c", subcore_axis_name="s")


@functools.partial(
    pl.kernel,
    out_type=jax.ShapeDtypeStruct((_N, _EMB), jnp.float32),
    mesh=_mesh,
    scratch_types=[
        pltpu.VMEM_SHARED((_VOCAB, _EMB), jnp.float32),
        pltpu.VMEM((_VOCAB, _EMB), jnp.float32),
        pltpu.VMEM((2, _CHUNK), jnp.int32),
        pltpu.VMEM((2, _CHUNK, _EMB), jnp.float32),
        pltpu.SemaphoreType.DMA((2,)),
        pltpu.SemaphoreType.DMA((2,)),
        pltpu.SemaphoreType.DMA((2,)),
    ],
    compiler_params=pltpu.CompilerParams(use_tc_tiling_on_sc=False,
                                         needs_layout_passes=False),
)
def _gather_kernel(ids_hbm, table_hbm, out_hbm, table_s, table_v, idx_v,
                   rows_v, sem_idx, sem_g, sem_w):
    wid = lax.axis_index("s") * _NC + lax.axis_index("c")
    base = wid * _N_PER_W

    @pl.when(lax.axis_index("s") == 0)
    def _():
        pltpu.sync_copy(table_hbm, table_s)
    pltpu.sync_copy(table_hbm, table_v)
    plsc.subcore_barrier()

    lane = lax.iota(jnp.int32, 16)

    for s in range(2):
        pltpu.async_copy(ids_hbm.at[pl.ds(base + s * _CHUNK, _CHUNK)],
                         idx_v.at[s], sem_idx.at[s])

    @pl.loop(0, _N_CHUNKS, step=2)
    def _steady(i):
        for s in range(2):
            c = i + s
            off = base + c * _CHUNK
            pltpu.make_async_copy(ids_hbm.at[pl.ds(off, _CHUNK)],
                                  idx_v.at[s], sem_idx.at[s]).wait()

            @pl.when(c >= 2)
            def _():
                pltpu.make_async_copy(
                    rows_v.at[s],
                    out_hbm.at[pl.ds(off - 2 * _CHUNK, _CHUNK)],
                    sem_w.at[s]).wait()

            pltpu.async_copy(table_s.at[idx_v.at[s, pl.ds(0, _Q)]],
                             rows_v.at[s, pl.ds(0, _Q)], sem_g.at[s])

            idx_ref = idx_v.at[s]
            rows_ref = rows_v.at[s]

            @plsc.parallel_loop(0, _RGROUPS)
            def _group(g):
                p = _Q + g * 16 + lane
                idvec = plsc.load_gather(idx_ref, [p])
                vals = [plsc.load_gather(table_v, [idvec, jnp.full((16,), e,
                                                                  jnp.int32)])
                        for e in range(_EMB)]
                for e in range(_EMB):
                    plsc.store_scatter(rows_ref,
                                       [p, jnp.full((16,), e, jnp.int32)],
                                       vals[e])

            pltpu.make_async_copy(table_s.at[idx_v.at[s, pl.ds(0, _Q)]],
                                  rows_v.at[s, pl.ds(0, _Q)],
                                  sem_g.at[s]).wait()
            pltpu.async_copy(rows_v.at[s], out_hbm.at[pl.ds(off, _CHUNK)],
                             sem_w.at[s])

            @pl.when(c + 2 < _N_CHUNKS)
            def _():
                pltpu.async_copy(
                    ids_hbm.at[pl.ds(off + 2 * _CHUNK, _CHUNK)],
                    idx_v.at[s], sem_idx.at[s])

    for s in range(2):
        off = base + (_N_CHUNKS - 2 + s) * _CHUNK
        pltpu.make_async_copy(rows_v.at[s], out_hbm.at[pl.ds(off, _CHUNK)],
                              sem_w.at[s]).wait()


def kernel(char_ids, char_embs):
    ids_flat = char_ids.reshape(_N)
    out = _gather_kernel(ids_flat, char_embs)
    return out.reshape(_B, _L, _EMB)


# speedup vs baseline: 2.7211x; 1.1641x over previous
import functools

import jax
import jax.numpy as jnp
from jax import lax
from jax.experimental import pallas as pl
from jax.experimental.pallas import tpu as pltpu
from jax.experimental.pallas import tpu_sc as plsc

_VOCAB = 256
_EMB = 32
_PAD = 33
_B = 4096
_L = 200
_N = _B * _L
_NC = 2
_NS = 16
_NW = _NC * _NS
_N_PER_W = _N // _NW
_CHUNK = 1600
_N_CHUNKS = _N_PER_W // _CHUNK
_GROUPS = _CHUNK // 16

_mesh = plsc.VectorSubcoreMesh(core_axis_name="c", subcore_axis_name="s")


@functools.partial(
    pl.kernel,
    out_type=jax.ShapeDtypeStruct((_N * _EMB,), jnp.float32),
    mesh=_mesh,
    scratch_types=[
        pltpu.VMEM((_VOCAB * _PAD,), jnp.float32),
        pltpu.VMEM((2, _CHUNK), jnp.int32),
        pltpu.VMEM((2, _CHUNK * _EMB), jnp.float32),
        pltpu.SemaphoreType.DMA((2,)),
        pltpu.SemaphoreType.DMA((2,)),
    ],
    compiler_params=pltpu.CompilerParams(use_tc_tiling_on_sc=False,
                                         needs_layout_passes=False),
)
def _gather_kernel(ids_hbm, table_hbm, out_hbm, table_v, idx_v, rows_v,
                   sem_idx, sem_w):
    wid = lax.axis_index("s") * _NC + lax.axis_index("c")
    base = wid * _N_PER_W

    pltpu.sync_copy(table_hbm, table_v)

    lane = lax.iota(jnp.int32, 16)

    for s in range(2):
        pltpu.async_copy(ids_hbm.at[pl.ds(base + s * _CHUNK, _CHUNK)],
                         idx_v.at[s], sem_idx.at[s])

    @pl.loop(0, _N_CHUNKS, step=2)
    def _steady(i):
        for s in range(2):
            c = i + s
            off = base + c * _CHUNK
            pltpu.make_async_copy(ids_hbm.at[pl.ds(off, _CHUNK)],
                                  idx_v.at[s], sem_idx.at[s]).wait()

            @pl.when(c >= 2)
            def _():
                pltpu.make_async_copy(
                    rows_v.at[s],
                    out_hbm.at[pl.ds((off - 2 * _CHUNK) * _EMB,
                                     _CHUNK * _EMB)],
                    sem_w.at[s]).wait()

            idx_ref = idx_v.at[s]
            rows_ref = rows_v.at[s]

            @plsc.parallel_loop(0, _GROUPS)
            def _group(g):
                idvec = plsc.load_gather(idx_ref, [g * 16 + lane])
                pbase = g * (16 * _EMB) + lane
                for j in range(16):
                    idj = idvec.at[jnp.full((16,), j, jnp.int32)].get(
                        mode="promise_in_bounds")
                    a0 = idj * _PAD + lane
                    v0 = plsc.load_gather(table_v, [a0])
                    v1 = plsc.load_gather(table_v, [a0 + 16])
                    st0 = pbase + j * _EMB
                    plsc.store_scatter(rows_ref, [st0], v0)
                    plsc.store_scatter(rows_ref, [st0 + 16], v1)

            pltpu.async_copy(rows_v.at[s],
                             out_hbm.at[pl.ds(off * _EMB, _CHUNK * _EMB)],
                             sem_w.at[s])

            @pl.when(c + 2 < _N_CHUNKS)
            def _():
                pltpu.async_copy(
                    ids_hbm.at[pl.ds(off + 2 * _CHUNK, _CHUNK)],
                    idx_v.at[s], sem_idx.at[s])

    for s in range(2):
        off = base + (_N_CHUNKS - 2 + s) * _CHUNK
        pltpu.make_async_copy(rows_v.at[s],
                              out_hbm.at[pl.ds(off * _EMB, _CHUNK * _EMB)],
                              sem_w.at[s]).wait()


def kernel(char_ids, char_embs):
    ids_flat = char_ids.reshape(_N)
    table_pad = jnp.pad(char_embs, ((0, 0), (0, _PAD - _EMB)))
    out = _gather_kernel(ids_flat, table_pad.reshape(_VOCAB * _PAD))
    return out.reshape(_B, _L, _EMB)
